# Initial kernel scaffold; baseline (speedup 1.0000x reference)
import jax
import jax.numpy as jnp
from jax.experimental import pallas as pl


def kernel(logits, labels):
    raise NotImplementedError("write your pallas kernel here")


# speedup vs baseline: 1.2932x; 1.2932x over previous
import jax
import jax.numpy as jnp
from jax.experimental import pallas as pl
from jax.experimental.pallas import tpu as pltpu

_B, _S, _V = 16, 512, 1000
_VP = 1024


def _rouge_body(logits_ref, labels_ref, out_ref):
    x = logits_ref[0]
    m = jnp.max(x, axis=1, keepdims=True)
    e = jnp.exp(x - m)
    denom = jnp.sum(e, axis=1, keepdims=True)
    p = e / denom

    labs = labels_ref[0]
    iota_v = jax.lax.broadcasted_iota(jnp.int32, (_S, _VP), 1)
    eq = (labs == iota_v).astype(jnp.float32)
    c = jnp.sum(eq, axis=0, keepdims=True)

    col_top = jnp.max(p, axis=0, keepdims=True)
    row_top = jnp.max(jnp.where(c > 0.0, p, -1.0), axis=1, keepdims=True)

    sel = jnp.logical_and(p == col_top, p == row_top).astype(jnp.float32)
    num = jnp.sum(p * sel * c)
    out_ref[...] = jnp.full((1, 1, 128), num * (2.0 / (2 * _S)), jnp.float32)


def kernel(logits, labels):
    logits_p = jnp.pad(logits, ((0, 0), (0, 0), (0, _VP - _V)),
                       constant_values=-1e30)
    labels3 = labels.reshape(_B, _S, 1)
    out = pl.pallas_call(
        _rouge_body,
        grid=(_B,),
        in_specs=[
            pl.BlockSpec((1, _S, _VP), lambda b: (b, 0, 0)),
            pl.BlockSpec((1, _S, 1), lambda b: (b, 0, 0)),
        ],
        out_specs=pl.BlockSpec((1, 1, 128), lambda b: (b, 0, 0)),
        out_shape=jax.ShapeDtypeStruct((_B, 1, 128), jnp.float32),
    )(logits_p, labels3)
    return out[:, 0, :1]


# speedup vs baseline: 1.8879x; 1.4599x over previous
import jax
import jax.numpy as jnp
from jax.experimental import pallas as pl
from jax.experimental.pallas import tpu as pltpu

_B, _S, _V = 16, 512, 1000
_VP = 1024


def _rouge_body(logits_ref, labels_ref, out_ref):
    x = logits_ref[0]
    m = jnp.max(x, axis=1, keepdims=True)
    e = jnp.exp(x - m)
    denom = jnp.sum(e, axis=1, keepdims=True)
    p = e * (1.0 / denom)

    labs = labels_ref[0]
    iota_v = jax.lax.broadcasted_iota(jnp.int32, (_S, _V), 1)
    eq = (labs == iota_v).astype(jnp.float32)
    c = jnp.sum(eq, axis=0, keepdims=True)

    col_top = jnp.max(p, axis=0, keepdims=True)
    row_top = jnp.max(jnp.where(c > 0.0, p, -1.0), axis=1, keepdims=True)

    sel = jnp.logical_and(p == col_top, p == row_top).astype(jnp.float32)
    num = jnp.sum(p * sel * c)
    out_ref[...] = jnp.full((1, 1, 128), num * (2.0 / (2 * _S)), jnp.float32)


def kernel(logits, labels):
    labels3 = labels.reshape(_B, _S, 1)
    out = pl.pallas_call(
        _rouge_body,
        grid=(_B,),
        in_specs=[
            pl.BlockSpec((1, _S, _V), lambda b: (b, 0, 0)),
            pl.BlockSpec((1, _S, 1), lambda b: (b, 0, 0)),
        ],
        out_specs=pl.BlockSpec((1, 1, 128), lambda b: (b, 0, 0)),
        out_shape=jax.ShapeDtypeStruct((_B, 1, 128), jnp.float32),
    )(logits, labels3)
    return out[:, 0, :1]
